# baseline (speedup 1.0000x reference)
import functools

import jax
import jax.numpy as jnp
from jax import lax
from jax.experimental import pallas as pl
from jax.experimental.pallas import tpu as pltpu
from jax.experimental.pallas import tpu_sc as plsc

_N_TOKENS = 1000000
_D_PROJ = 128
_CUT0 = 20000
_CUT1 = 100000
_N2 = _N_TOKENS - _CUT1
_SCALE = float(_D_PROJ) ** 0.5

_T = 819200
_NW = 32
_PER_W = _T // _NW
_CH = 128
_NCK = _PER_W // _CH
_EXC_SZ = _PER_W + _CH

_PP_BLK = 4000
_PP_N = _CUT1 // _PP_BLK
_PP_B0 = _CUT0 // _PP_BLK

_MM_BLK = 8192
_MM_N = _T // _MM_BLK

_SC_PARAMS = pltpu.CompilerParams(needs_layout_passes=False,
                                  use_tc_tiling_on_sc=False)

_GD = lax.GatherDimensionNumbers(
    offset_dims=(), collapsed_slice_dims=(0,), start_index_map=(0,))


def _xgather(x, inds):
    return lax.gather(x, inds[:, None], _GD, slice_sizes=(1,),
                      mode=lax.GatherScatterMode.PROMISE_IN_BOUNDS)


def _pp01_body(emb0_ref, emb1_ref, p0_ref, p1_ref, out_ref):
    g = pl.program_id(0)

    @pl.when(g < _PP_B0)
    def _():
        out_ref[...] = lax.dot_general(
            emb0_ref[...], p0_ref[...] * _SCALE,
            (((1,), (1,)), ((), ())), preferred_element_type=jnp.float32)

    @pl.when(g >= _PP_B0)
    def _():
        out_ref[...] = lax.dot_general(
            emb1_ref[...], p1_ref[...] * _SCALE,
            (((1,), (1,)), ((), ())), preferred_element_type=jnp.float32)


def _preproject01(emb0, emb1, proj0, proj1):
    return pl.pallas_call(
        _pp01_body,
        grid=(_PP_N,),
        in_specs=[
            pl.BlockSpec((_PP_BLK, 128),
                         lambda g: (jnp.minimum(g, _PP_B0 - 1), 0)),
            pl.BlockSpec((_PP_BLK, 32),
                         lambda g: (jnp.clip(g - _PP_B0, 0, _PP_N - _PP_B0 - 1), 0)),
            pl.BlockSpec((128, 128), lambda g: (0, 0)),
            pl.BlockSpec((128, 32), lambda g: (0, 0)),
        ],
        out_specs=pl.BlockSpec((_PP_BLK, 128), lambda g: (g, 0)),
        out_shape=jax.ShapeDtypeStruct((_CUT1, _D_PROJ), jnp.float32),
    )(emb0, emb1, proj0, proj1)


def _sc_route(idx, emb2):
    mesh = plsc.VectorSubcoreMesh(core_axis_name="c", subcore_axis_name="s")
    info = plsc.get_sparse_core_info()

    @functools.partial(
        pl.kernel,
        mesh=mesh,
        compiler_params=_SC_PARAMS,
        out_type=[
            jax.ShapeDtypeStruct((_T, 8), jnp.float32),
            jax.ShapeDtypeStruct((_NW, _EXC_SZ), jnp.int32),
            jax.ShapeDtypeStruct((_NW, _NCK, _CH), jnp.int32),
            jax.ShapeDtypeStruct((_NW, 16), jnp.int32),
        ],
        scratch_types=[
            pltpu.VMEM((_PER_W,), jnp.int32),
            pltpu.VMEM((_EXC_SZ,), jnp.int32),
            pltpu.VMEM((_EXC_SZ,), jnp.int32),
            pltpu.VMEM((_NCK, _CH), jnp.int32),
            pltpu.VMEM((16,), jnp.int32),
            pltpu.VMEM((_CH,), jnp.int32),
            pltpu.VMEM((_CH,), jnp.int32),
            pltpu.VMEM((_CH, 8), jnp.float32),
            pltpu.VMEM((_CH, 8), jnp.float32),
            pltpu.SemaphoreType.DMA,
            pltpu.SemaphoreType.DMA,
            pltpu.SemaphoreType.DMA,
            pltpu.SemaphoreType.DMA,
        ],
    )
    def route(idx_hbm, emb2_hbm, buf8_hbm, exci_hbm, posd_hbm, cnt_hbm,
              idx_v, exci_v, excp_v, posd_v, cnt_v, g8a, g8b, r8a, r8b,
              sg0, sg1, ss0, ss1):
        wid = lax.axis_index("s") * info.num_cores + lax.axis_index("c")
        base = wid * _PER_W
        pltpu.sync_copy(idx_hbm.at[pl.ds(base, _PER_W)], idx_v)

        g8 = (g8a, g8b)
        r8 = (r8a, r8b)
        sgs = (sg0, sg1)
        sss = (ss0, ss1)

        iota = lax.iota(jnp.int32, 16)

        def do_compute(j, g8buf, offv):
            for k in range(8):
                s = j * _CH + k * 16
                iv = idx_v[pl.ds(s, 16)]
                g8buf[pl.ds(k * 16, 16)] = jnp.clip(iv - _CUT1, 0, _N2 - 1)
                m01 = jnp.where(iv < _CUT1, 1, 0)
                p = m01
                for sft in (1, 2, 4, 8):
                    sh = _xgather(p, jnp.maximum(iota - sft, 0))
                    p = p + jnp.where(iota >= sft, sh, 0)
                cntg = _xgather(p, jnp.full((16,), 15, jnp.int32))
                dest = jnp.where(m01 == 1, offv + p - m01, offv + cntg)
                pos = (base + s) + iota
                plsc.store_scatter(exci_v, [dest], iv)
                plsc.store_scatter(excp_v, [dest], pos)
                offv = offv + cntg
            return offv

        def fire_gather(g8buf, rbuf, sem):
            pltpu.async_copy(emb2_hbm.at[g8buf], rbuf, sem)

        def wait_gather(b):
            pltpu.make_async_copy(emb2_hbm.at[g8[b]], r8[b], sgs[b]).wait()

        def fire_store(j, rbuf, sem):
            pltpu.async_copy(rbuf, buf8_hbm.at[pl.ds(base + j * _CH, _CH)],
                             sem)

        def wait_store(b):
            pltpu.make_async_copy(r8[b], buf8_hbm.at[pl.ds(base, _CH)],
                                  sss[b]).wait()

        off0 = do_compute(0, g8a, jnp.zeros((16,), jnp.int32))
        fire_gather(g8a, r8a, sg0)
        off0 = do_compute(1, g8b, off0)
        fire_gather(g8b, r8b, sg1)
        wait_gather(0)
        fire_store(0, r8a, ss0)

        def pair(p, off):
            def one(j, bj, off):
                bn = 1 - bj
                off = do_compute(j + 1, g8[bn], off)
                wait_store(bn)
                fire_gather(g8[bn], r8[bn], sgs[bn])
                wait_gather(bj)
                fire_store(j, r8[bj], sss[bj])
                return off
            off = one(2 * p + 1, 1, off)
            off = one(2 * p + 2, 0, off)
            return off

        cntv = lax.fori_loop(0, (_NCK - 2) // 2, pair, off0)
        cnt = jnp.max(cntv)
        wait_gather(1)
        fire_store(_NCK - 1, r8[1], sss[1])
        wait_store(0)
        wait_store(1)

        @pl.when(cnt > 0)
        def _():
            lastv = plsc.load_gather(
                exci_v, [jnp.full((16,), cnt - 1, jnp.int32)])
            for k in range(8):
                exci_v[pl.ds(cnt + k * 16, 16)] = lastv

        nch = (cnt + _CH - 1) // _CH

        def rep(j2, _):
            for k in range(8):
                src = jnp.minimum(j2 * _CH + k * 16 + lax.iota(jnp.int32, 16),
                                  cnt - 1)
                posd_v[j2, pl.ds(k * 16, 16)] = plsc.load_gather(excp_v, [src])
            return 0

        lax.fori_loop(0, nch, rep, 0)

        cnt_v[...] = jnp.full((16,), cnt, jnp.int32)
        pltpu.sync_copy(exci_v, exci_hbm.at[wid])
        pltpu.sync_copy(posd_v, posd_hbm.at[wid])
        pltpu.sync_copy(cnt_v, cnt_hbm.at[wid])

    return route(idx, emb2)


def _mm_body(b8_ref, p2_ref, out_ref):
    out_ref[...] = lax.dot_general(
        b8_ref[...], p2_ref[...] * _SCALE,
        (((1,), (1,)), ((), ())), preferred_element_type=jnp.float32)


def _base_matmul(buf8, proj2):
    return pl.pallas_call(
        _mm_body,
        grid=(_MM_N,),
        in_specs=[
            pl.BlockSpec((_MM_BLK, 8), lambda g: (g, 0)),
            pl.BlockSpec((128, 8), lambda g: (0, 0)),
        ],
        out_specs=pl.BlockSpec((_MM_BLK, 128), lambda g: (g, 0)),
        out_shape=jax.ShapeDtypeStruct((_T, _D_PROJ), jnp.float32),
    )(buf8, proj2)


def _sc_fixup(pt01, exci, posd, cnts, out_ref):
    mesh = plsc.VectorSubcoreMesh(core_axis_name="c", subcore_axis_name="s")
    info = plsc.get_sparse_core_info()

    @functools.partial(
        pl.kernel,
        mesh=mesh,
        compiler_params=_SC_PARAMS,
        out_type=(),
        scratch_types=[
            pltpu.VMEM((_EXC_SZ,), jnp.int32),
            pltpu.VMEM((_NCK, _CH), jnp.int32),
            pltpu.VMEM((16,), jnp.int32),
            pltpu.VMEM((_CH, _D_PROJ), jnp.float32),
            pltpu.SemaphoreType.DMA,
            pltpu.SemaphoreType.DMA,
        ],
    )
    def fixup(pt01_hbm, exci_hbm, posd_hbm, cnt_hbm, out_hbm,
              exci_v, posd_v, cnt_v, ra, sg, ss):
        wid = lax.axis_index("s") * info.num_cores + lax.axis_index("c")
        pltpu.sync_copy(cnt_hbm.at[wid], cnt_v)
        cnt = jnp.max(cnt_v[...])
        nch = (cnt + _CH - 1) // _CH
        pltpu.sync_copy(exci_hbm.at[wid], exci_v)
        pltpu.sync_copy(posd_hbm.at[wid], posd_v)

        def body(j, _):
            pltpu.async_copy(
                pt01_hbm.at[exci_v.at[pl.ds(j * _CH, _CH)]], ra, sg)
            pltpu.make_async_copy(
                pt01_hbm.at[exci_v.at[pl.ds(0, _CH)]], ra, sg).wait()
            pltpu.async_copy(ra, out_hbm.at[posd_v.at[j]], ss)
            pltpu.make_async_copy(ra, out_hbm.at[posd_v.at[0]], ss).wait()
            return 0

        lax.fori_loop(0, nch, body, 0)

    fixup(pt01, exci, posd, cnts, out_ref)


def kernel(indices, emb0, emb1, emb2, proj0, proj1, proj2):
    idx = indices.reshape(-1)
    pt01 = _preproject01(emb0, emb1, proj0, proj1)
    buf8, exci, posd, cnts = _sc_route(idx, emb2)
    base_out = _base_matmul(buf8, proj2)
    out_ref = jax.new_ref(base_out)
    _sc_fixup(pt01, exci, posd, cnts, out_ref)
    return out_ref[...].reshape(indices.shape + (_D_PROJ,))


# speedup vs baseline: 1.6142x; 1.6142x over previous
import functools

import jax
import jax.numpy as jnp
from jax import lax
from jax.experimental import pallas as pl
from jax.experimental.pallas import tpu as pltpu
from jax.experimental.pallas import tpu_sc as plsc

_N_TOKENS = 1000000
_D_PROJ = 128
_CUT0 = 20000
_CUT1 = 100000
_SCALE = float(_D_PROJ) ** 0.5

_ROWS_PER_BLK = 10000
_N_BLKS = _N_TOKENS // _ROWS_PER_BLK
_B0 = _CUT0 // _ROWS_PER_BLK
_B1 = _CUT1 // _ROWS_PER_BLK

_CHUNK = 128


def _preproject_body(emb0_ref, emb1_ref, emb2_ref, p0_ref, p1_ref, p2_ref,
                     out_ref):
    g = pl.program_id(0)

    @pl.when(g < _B0)
    def _():
        out_ref[...] = lax.dot_general(
            emb0_ref[...], p0_ref[...] * _SCALE,
            (((1,), (1,)), ((), ())),
            preferred_element_type=jnp.float32)

    @pl.when((g >= _B0) & (g < _B1))
    def _():
        out_ref[...] = lax.dot_general(
            emb1_ref[...], p1_ref[...] * _SCALE,
            (((1,), (1,)), ((), ())),
            preferred_element_type=jnp.float32)

    @pl.when(g >= _B1)
    def _():
        out_ref[...] = lax.dot_general(
            emb2_ref[...], p2_ref[...] * _SCALE,
            (((1,), (1,)), ((), ())),
            preferred_element_type=jnp.float32)


def _preproject(emb0, emb1, emb2, proj0, proj1, proj2):
    return pl.pallas_call(
        _preproject_body,
        grid=(_N_BLKS,),
        in_specs=[
            pl.BlockSpec((_ROWS_PER_BLK, 128),
                         lambda g: (jnp.minimum(g, _B0 - 1), 0)),
            pl.BlockSpec((_ROWS_PER_BLK, 32),
                         lambda g: (jnp.clip(g - _B0, 0, _B1 - _B0 - 1), 0)),
            pl.BlockSpec((_ROWS_PER_BLK, 8),
                         lambda g: (jnp.clip(g - _B1, 0, _N_BLKS - _B1 - 1), 0)),
            pl.BlockSpec((128, 128), lambda g: (0, 0)),
            pl.BlockSpec((128, 32), lambda g: (0, 0)),
            pl.BlockSpec((128, 8), lambda g: (0, 0)),
        ],
        out_specs=pl.BlockSpec((_ROWS_PER_BLK, 128), lambda g: (g, 0)),
        out_shape=jax.ShapeDtypeStruct((_N_TOKENS, _D_PROJ), jnp.float32),
    )(emb0, emb1, emb2, proj0, proj1, proj2)


def _gather(pt, idx):
    n_tok = idx.shape[0]
    info = plsc.get_sparse_core_info()
    nw = info.num_cores * info.num_subcores
    per_w = n_tok // nw
    n_chunks = per_w // _CHUNK
    mesh = plsc.VectorSubcoreMesh(core_axis_name="c", subcore_axis_name="s")

    @functools.partial(
        pl.kernel,
        mesh=mesh,
        out_type=jax.ShapeDtypeStruct((n_tok, _D_PROJ), jnp.float32),
        scratch_types=[
            pltpu.VMEM((per_w,), jnp.int32),
            pltpu.VMEM((_CHUNK, _D_PROJ), jnp.float32),
            pltpu.VMEM((_CHUNK, _D_PROJ), jnp.float32),
            pltpu.SemaphoreType.DMA,
            pltpu.SemaphoreType.DMA,
            pltpu.SemaphoreType.DMA,
            pltpu.SemaphoreType.DMA,
        ],
    )
    def sc_gather(pt_hbm, idx_hbm, out_hbm, idx_v, row0, row1, sg0, sg1,
                  ss0, ss1):
        wid = lax.axis_index("s") * info.num_cores + lax.axis_index("c")
        base = wid * per_w
        pltpu.sync_copy(idx_hbm.at[pl.ds(base, per_w)], idx_v)

        rows = (row0, row1)
        sgs = (sg0, sg1)
        sss = (ss0, ss1)

        def start_gather(j, b):
            pltpu.async_copy(
                pt_hbm.at[idx_v.at[pl.ds(j * _CHUNK, _CHUNK)]], rows[b],
                sgs[b])

        def start_store(j, b):
            pltpu.async_copy(
                rows[b], out_hbm.at[pl.ds(base + j * _CHUNK, _CHUNK)],
                sss[b])

        def wait_gather(b):
            pltpu.make_async_copy(pt_hbm.at[idx_v.at[pl.ds(0, _CHUNK)]],
                                  rows[b], sgs[b]).wait()

        def wait_store(b):
            pltpu.make_async_copy(rows[b],
                                  out_hbm.at[pl.ds(base, _CHUNK)],
                                  sss[b]).wait()

        start_gather(0, 0)
        start_gather(1, 1)
        wait_gather(0)
        start_store(0, 0)

        def body(jj, _):
            def step(j, b):
                nb = 1 - b
                wait_store(nb)
                start_gather(j + 1, nb)
                wait_gather(b)
                start_store(j, b)
            step(2 * jj + 1, 1)
            step(2 * jj + 2, 0)
            return 0

        lax.fori_loop(0, (n_chunks - 2) // 2, body, 0)

        wait_gather(1)
        start_store(n_chunks - 1, 1)
        wait_store(0)
        wait_store(1)

    return sc_gather(pt, idx)


def kernel(indices, emb0, emb1, emb2, proj0, proj1, proj2):
    pt = _preproject(emb0, emb1, emb2, proj0, proj1, proj2)
    idx = indices.reshape(-1)
    out = _gather(pt, idx)
    return out.reshape(indices.shape + (_D_PROJ,))
